# baseline (speedup 1.0000x reference)
import functools

import jax
from jax import lax
import jax.numpy as jnp
from jax.experimental import pallas as pl
from jax.experimental.pallas import tpu as pltpu
from jax.experimental.pallas import tpu_sc as plsc

_B, _D, _N, _TOPK = 4096, 2048, 64, 3
_TB1 = 1024
_TB3 = 512
_NW = 32
_RW = _B // _NW
_L = 16


def _k1_body(x_ref, protos_ref, w1xt_ref, w1p_ref, b1_ref,
             simsT_ref, xw1_ref, ppw1b_ref):
    x = x_ref[...]
    protos = protos_ref[...]

    pn = protos / jnp.maximum(
        jnp.sqrt(jnp.sum(protos * protos, axis=1, keepdims=True)), 1e-12)
    nc = jnp.maximum(
        jnp.sqrt(jnp.sum(x * x, axis=1, keepdims=True)), 1e-12)
    rn = x / nc

    rhs = jnp.concatenate([pn, w1xt_ref[...]], axis=0)
    zt = jax.lax.dot_general(
        rhs, rn, (((1,), (1,)), ((), ())),
        preferred_element_type=jnp.float32)

    eye = (lax.broadcasted_iota(jnp.int32, (_TB1, _TB1), 0) ==
           lax.broadcasted_iota(jnp.int32, (_TB1, _TB1), 1)).astype(jnp.float32)
    ncT = jax.lax.dot_general(nc, eye, (((0,), (0,)), ((), ())),
                              preferred_element_type=jnp.float32)

    simsT_ref[...] = zt[:_N, :]
    xw1_ref[...] = zt[_N:, :] * ncT

    @pl.when(pl.program_id(0) == 0)
    def _():
        ppw1b_ref[...] = jax.lax.dot_general(
            protos, w1p_ref[...], (((1,), (0,)), ((), ())),
            preferred_element_type=jnp.float32) + b1_ref[...]


def _k1(x, protos, w1xt, w1p, b1r):
    return pl.pallas_call(
        _k1_body,
        grid=(_B // _TB1,),
        in_specs=[
            pl.BlockSpec((_TB1, _D), lambda i: (i, 0)),
            pl.BlockSpec((_N, _D), lambda i: (0, 0)),
            pl.BlockSpec((32, _D), lambda i: (0, 0)),
            pl.BlockSpec((_D, 32), lambda i: (0, 0)),
            pl.BlockSpec((1, 32), lambda i: (0, 0)),
        ],
        out_specs=[
            pl.BlockSpec((_N, _TB1), lambda i: (0, i)),
            pl.BlockSpec((32, _TB1), lambda i: (0, i)),
            pl.BlockSpec((_N, 32), lambda i: (0, 0)),
        ],
        out_shape=[
            jax.ShapeDtypeStruct((_N, _B), jnp.float32),
            jax.ShapeDtypeStruct((32, _B), jnp.float32),
            jax.ShapeDtypeStruct((_N, 32), jnp.float32),
        ],
        compiler_params=pltpu.CompilerParams(
            dimension_semantics=("arbitrary",)),
    )(x, protos, w1xt, w1p, b1r)


@functools.partial(
    pl.kernel,
    out_type=[
        jax.ShapeDtypeStruct((4, _B), jnp.int32),
        jax.ShapeDtypeStruct((4, _B), jnp.float32),
    ],
    mesh=plsc.VectorSubcoreMesh(core_axis_name="c", subcore_axis_name="s"),
    scratch_types=[
        pltpu.VMEM((_N, _RW), jnp.float32),
        pltpu.VMEM((4, _RW), jnp.int32),
        pltpu.VMEM((4, _RW), jnp.float32),
    ],
)
def _k2_sc(simsT_hbm, idx_hbm, wts_hbm, sims_v, idx_v, wts_v):
    wid = lax.axis_index("s") * 2 + lax.axis_index("c")
    base = wid * _RW
    pltpu.sync_copy(simsT_hbm.at[:, pl.ds(base, _RW)], sims_v)

    def group_body(g, _):
        col0 = pl.multiple_of(g * _L, _L)

        def body(n, carry):
            m1, m2, m3, i1, i2, i3 = carry
            c = sims_v[n, pl.ds(col0, _L)]
            nvec = jnp.full((_L,), 0, jnp.int32) + n
            is1 = c > m1
            is2 = c > m2
            is3 = c > m3
            m3n = jnp.where(is2, m2, jnp.where(is3, c, m3))
            i3n = jnp.where(is2, i2, jnp.where(is3, nvec, i3))
            m2n = jnp.where(is1, m1, jnp.where(is2, c, m2))
            i2n = jnp.where(is1, i1, jnp.where(is2, nvec, i2))
            m1n = jnp.where(is1, c, m1)
            i1n = jnp.where(is1, nvec, i1)
            return (m1n, m2n, m3n, i1n, i2n, i3n)

        neg = jnp.full((_L,), -jnp.inf, jnp.float32)
        zi = jnp.zeros((_L,), jnp.int32)
        m1, m2, m3, i1, i2, i3 = lax.fori_loop(
            0, _N, body, (neg, neg, neg, zi, zi, zi))

        e2 = jnp.exp(5.0 * (m2 - m1))
        e3 = jnp.exp(5.0 * (m3 - m1))
        den = 1.0 + e2 + e3
        idx_v[0, pl.ds(col0, _L)] = i1
        idx_v[1, pl.ds(col0, _L)] = i2
        idx_v[2, pl.ds(col0, _L)] = i3
        idx_v[3, pl.ds(col0, _L)] = zi
        wts_v[0, pl.ds(col0, _L)] = 1.0 / den
        wts_v[1, pl.ds(col0, _L)] = e2 / den
        wts_v[2, pl.ds(col0, _L)] = e3 / den
        wts_v[3, pl.ds(col0, _L)] = jnp.zeros((_L,), jnp.float32)
        return 0

    lax.fori_loop(0, _RW // _L, group_body, 0)

    pltpu.sync_copy(idx_v, idx_hbm.at[:, pl.ds(base, _RW)])
    pltpu.sync_copy(wts_v, wts_hbm.at[:, pl.ds(base, _RW)])


def _k3_body(xw1_ref, idx_ref, wts_ref, ppw1b_ref, w2_ref, b2_ref, out_ref):
    xw1 = xw1_ref[...]
    ppw1b = ppw1b_ref[...]
    iotaN = lax.broadcasted_iota(jnp.int32, (_N, _TB3), 0)
    hsum = jnp.zeros_like(xw1)
    for k in range(_TOPK):
        idx_k = idx_ref[k:k + 1, :]
        onehot = (iotaN == idx_k).astype(jnp.float32)
        pk = jax.lax.dot_general(ppw1b, onehot, (((0,), (0,)), ((), ())),
                                 preferred_element_type=jnp.float32)
        hsum = hsum + wts_ref[k:k + 1, :] * jnp.maximum(xw1 + pk, 0.0)
    outT = jax.lax.dot_general(
        w2_ref[...], hsum, (((0,), (0,)), ((), ())),
        preferred_element_type=jnp.float32) + b2_ref[...]
    eye = (lax.broadcasted_iota(jnp.int32, (_TB3, _TB3), 0) ==
           lax.broadcasted_iota(jnp.int32, (_TB3, _TB3), 1)).astype(jnp.float32)
    out_ref[...] = jax.lax.dot_general(
        eye, outT, (((1,), (1,)), ((), ())),
        preferred_element_type=jnp.float32)


def _k3(xw1T, idx4, wts4, ppw1b, W2, b2r):
    return pl.pallas_call(
        _k3_body,
        grid=(_B // _TB3,),
        in_specs=[
            pl.BlockSpec((32, _TB3), lambda i: (0, i)),
            pl.BlockSpec((4, _TB3), lambda i: (0, i)),
            pl.BlockSpec((4, _TB3), lambda i: (0, i)),
            pl.BlockSpec((_N, 32), lambda i: (0, 0)),
            pl.BlockSpec((32, 16), lambda i: (0, 0)),
            pl.BlockSpec((16, 1), lambda i: (0, 0)),
        ],
        out_specs=pl.BlockSpec((_TB3, 16), lambda i: (i, 0)),
        out_shape=jax.ShapeDtypeStruct((_B, 16), jnp.float32),
        compiler_params=pltpu.CompilerParams(
            dimension_semantics=("arbitrary",)),
    )(xw1T, idx4, wts4, ppw1b, W2, b2r)


@jax.jit
def kernel(regime_vector, pattern_prototypes, W1, b1, W2, b2):
    w1xt = W1[:_D].T
    w1p = W1[_D:]
    b1r = b1.reshape(1, 32)
    b2c = b2.reshape(16, 1)
    simsT, xw1T, ppw1b = _k1(regime_vector, pattern_prototypes, w1xt, w1p, b1r)
    idx4, wts4 = _k2_sc(simsT)
    return _k3(xw1T, idx4, wts4, ppw1b, W2, b2c)


# speedup vs baseline: 1.1914x; 1.1914x over previous
import functools

import jax
from jax import lax
import jax.numpy as jnp
from jax.experimental import pallas as pl
from jax.experimental.pallas import tpu as pltpu
from jax.experimental.pallas import tpu_sc as plsc

_B, _D, _N, _TOPK = 4096, 2048, 64, 3
_TB1 = 1024
_TB3 = 2048
_NW = 32
_RW = _B // _NW
_L = 16


def _k1_body(x_ref, protos_ref, w1xt_ref, w1p_ref, b1_ref,
             simsT_ref, xw1_ref, ppw1b_ref):
    x = x_ref[...]
    protos = protos_ref[...]

    pn = protos / jnp.maximum(
        jnp.sqrt(jnp.sum(protos * protos, axis=1, keepdims=True)), 1e-12)
    nc = jnp.maximum(
        jnp.sqrt(jnp.sum(x * x, axis=1, keepdims=True)), 1e-12)
    rn = x / nc

    rhs = jnp.concatenate([pn, w1xt_ref[...]], axis=0)
    zt = jax.lax.dot_general(
        rhs, rn, (((1,), (1,)), ((), ())),
        preferred_element_type=jnp.float32)

    eye = (lax.broadcasted_iota(jnp.int32, (_TB1, _TB1), 0) ==
           lax.broadcasted_iota(jnp.int32, (_TB1, _TB1), 1)).astype(jnp.float32)
    ncT = jax.lax.dot_general(nc, eye, (((0,), (0,)), ((), ())),
                              preferred_element_type=jnp.float32)

    simsT_ref[...] = zt[:_N, :]
    xw1_ref[...] = zt[_N:, :] * ncT

    @pl.when(pl.program_id(0) == 0)
    def _():
        ppw1b_ref[...] = jax.lax.dot_general(
            protos, w1p_ref[...], (((1,), (0,)), ((), ())),
            preferred_element_type=jnp.float32) + b1_ref[...]


def _k1(x, protos, w1xt, w1p, b1r):
    return pl.pallas_call(
        _k1_body,
        grid=(_B // _TB1,),
        in_specs=[
            pl.BlockSpec((_TB1, _D), lambda i: (i, 0)),
            pl.BlockSpec((_N, _D), lambda i: (0, 0)),
            pl.BlockSpec((32, _D), lambda i: (0, 0)),
            pl.BlockSpec((_D, 32), lambda i: (0, 0)),
            pl.BlockSpec((1, 32), lambda i: (0, 0)),
        ],
        out_specs=[
            pl.BlockSpec((_N, _TB1), lambda i: (0, i)),
            pl.BlockSpec((32, _TB1), lambda i: (0, i)),
            pl.BlockSpec((_N, 32), lambda i: (0, 0)),
        ],
        out_shape=[
            jax.ShapeDtypeStruct((_N, _B), jnp.float32),
            jax.ShapeDtypeStruct((32, _B), jnp.float32),
            jax.ShapeDtypeStruct((_N, 32), jnp.float32),
        ],
        compiler_params=pltpu.CompilerParams(
            dimension_semantics=("arbitrary",)),
    )(x, protos, w1xt, w1p, b1r)


@functools.partial(
    pl.kernel,
    out_type=[
        jax.ShapeDtypeStruct((4, _B), jnp.int32),
        jax.ShapeDtypeStruct((4, _B), jnp.float32),
    ],
    mesh=plsc.VectorSubcoreMesh(core_axis_name="c", subcore_axis_name="s"),
    scratch_types=[
        pltpu.VMEM((_N, _RW), jnp.float32),
        pltpu.VMEM((4, _RW), jnp.int32),
        pltpu.VMEM((4, _RW), jnp.float32),
    ],
)
def _k2_sc(simsT_hbm, idx_hbm, wts_hbm, sims_v, idx_v, wts_v):
    wid = lax.axis_index("s") * 2 + lax.axis_index("c")
    base = wid * _RW
    pltpu.sync_copy(simsT_hbm.at[:, pl.ds(base, _RW)], sims_v)

    def group_body(g, _):
        col0 = pl.multiple_of(g * _L, _L)

        def body(n, carry):
            m1, m2, m3, i1, i2, i3 = carry
            c = sims_v[n, pl.ds(col0, _L)]
            nvec = jnp.full((_L,), 0, jnp.int32) + n
            is1 = c > m1
            is2 = c > m2
            is3 = c > m3
            m3n = jnp.where(is2, m2, jnp.where(is3, c, m3))
            i3n = jnp.where(is2, i2, jnp.where(is3, nvec, i3))
            m2n = jnp.where(is1, m1, jnp.where(is2, c, m2))
            i2n = jnp.where(is1, i1, jnp.where(is2, nvec, i2))
            m1n = jnp.where(is1, c, m1)
            i1n = jnp.where(is1, nvec, i1)
            return (m1n, m2n, m3n, i1n, i2n, i3n)

        neg = jnp.full((_L,), -jnp.inf, jnp.float32)
        zi = jnp.zeros((_L,), jnp.int32)
        m1, m2, m3, i1, i2, i3 = lax.fori_loop(
            0, _N, body, (neg, neg, neg, zi, zi, zi))

        e2 = jnp.exp(5.0 * (m2 - m1))
        e3 = jnp.exp(5.0 * (m3 - m1))
        den = 1.0 + e2 + e3
        idx_v[0, pl.ds(col0, _L)] = i1
        idx_v[1, pl.ds(col0, _L)] = i2
        idx_v[2, pl.ds(col0, _L)] = i3
        idx_v[3, pl.ds(col0, _L)] = zi
        wts_v[0, pl.ds(col0, _L)] = 1.0 / den
        wts_v[1, pl.ds(col0, _L)] = e2 / den
        wts_v[2, pl.ds(col0, _L)] = e3 / den
        wts_v[3, pl.ds(col0, _L)] = jnp.zeros((_L,), jnp.float32)
        return 0

    lax.fori_loop(0, _RW // _L, group_body, 0)

    pltpu.sync_copy(idx_v, idx_hbm.at[:, pl.ds(base, _RW)])
    pltpu.sync_copy(wts_v, wts_hbm.at[:, pl.ds(base, _RW)])


def _k3_body(xw1_ref, idx_ref, wts_ref, ppw1b_ref, w2_ref, b2_ref, out_ref):
    xw1 = xw1_ref[...]
    ppw1b = ppw1b_ref[...]
    iotaN = lax.broadcasted_iota(jnp.int32, (_N, _TB3), 0)
    hsum = jnp.zeros_like(xw1)
    for k in range(_TOPK):
        idx_k = idx_ref[k:k + 1, :]
        onehot = (iotaN == idx_k).astype(jnp.float32)
        pk = jax.lax.dot_general(ppw1b, onehot, (((0,), (0,)), ((), ())),
                                 preferred_element_type=jnp.float32)
        hsum = hsum + wts_ref[k:k + 1, :] * jnp.maximum(xw1 + pk, 0.0)
    out_ref[...] = jax.lax.dot_general(
        w2_ref[...], hsum, (((0,), (0,)), ((), ())),
        preferred_element_type=jnp.float32) + b2_ref[...]


def _k3(xw1T, idx4, wts4, ppw1b, W2, b2r):
    return pl.pallas_call(
        _k3_body,
        grid=(_B // _TB3,),
        in_specs=[
            pl.BlockSpec((32, _TB3), lambda i: (0, i)),
            pl.BlockSpec((4, _TB3), lambda i: (0, i)),
            pl.BlockSpec((4, _TB3), lambda i: (0, i)),
            pl.BlockSpec((_N, 32), lambda i: (0, 0)),
            pl.BlockSpec((32, 16), lambda i: (0, 0)),
            pl.BlockSpec((16, 1), lambda i: (0, 0)),
        ],
        out_specs=pl.BlockSpec((16, _TB3), lambda i: (0, i)),
        out_shape=jax.ShapeDtypeStruct((16, _B), jnp.float32),
        compiler_params=pltpu.CompilerParams(
            dimension_semantics=("arbitrary",)),
    )(xw1T, idx4, wts4, ppw1b, W2, b2r)


@jax.jit
def kernel(regime_vector, pattern_prototypes, W1, b1, W2, b2):
    w1xt = W1[:_D].T
    w1p = W1[_D:]
    b1r = b1.reshape(1, 32)
    b2c = b2.reshape(16, 1)
    simsT, xw1T, ppw1b = _k1(regime_vector, pattern_prototypes, w1xt, w1p, b1r)
    idx4, wts4 = _k2_sc(simsT)
    outT = _k3(xw1T, idx4, wts4, ppw1b, W2, b2c)
    return outT.T
